# baseline (speedup 1.0000x reference)
import functools

import jax
import jax.numpy as jnp
from jax import lax
from jax.experimental import pallas as pl
from jax.experimental.pallas import tpu as pltpu
from jax.experimental.pallas import tpu_sc as plsc

HIDDEN = 128
EPS = 1e-05

NC = 2
NS = 16
L = 16
NW = NC * NS

B_TOTAL = 4 * 8192
B_PER_W = B_TOTAL // NW
CH = 128
NCHUNK = B_PER_W // CH
NVREG = HIDDEN // L
NBUF = 4
PREFETCH = 2


def _make_kernel():
    mesh = plsc.VectorSubcoreMesh(core_axis_name="c", subcore_axis_name="s")

    @functools.partial(
        pl.kernel,
        mesh=mesh,
        out_type=jax.ShapeDtypeStruct((B_TOTAL, HIDDEN), jnp.float32),
        scratch_types=[
            pltpu.VMEM((B_PER_W,), jnp.int32),
            pltpu.VMEM((CH, HIDDEN), jnp.float32),
            pltpu.VMEM((CH, HIDDEN), jnp.float32),
            pltpu.VMEM((CH, HIDDEN), jnp.float32),
            pltpu.VMEM((CH, HIDDEN), jnp.float32),
            pltpu.VMEM((HIDDEN,), jnp.float32),
            pltpu.SemaphoreType.DMA,
            pltpu.SemaphoreType.DMA,
            pltpu.SemaphoreType.DMA,
            pltpu.SemaphoreType.DMA,
            pltpu.SemaphoreType.DMA,
            pltpu.SemaphoreType.DMA,
            pltpu.SemaphoreType.DMA,
            pltpu.SemaphoreType.DMA,
        ],
        compiler_params=pltpu.CompilerParams(needs_layout_passes=False),
    )
    def kern(ids_hbm, table_hbm, w_hbm, out_hbm, idx_v, rows0, rows1, rows2,
             rows3, w_v, gsem0, gsem1, gsem2, gsem3, wsem0, wsem1, wsem2,
             wsem3):
        wid = lax.axis_index("s") * NC + lax.axis_index("c")
        base = wid * B_PER_W
        pltpu.sync_copy(w_hbm, w_v)
        wv = [w_v[pl.ds(L * j, L)] for j in range(NVREG)]
        rows = (rows0, rows1, rows2, rows3)
        gsem = (gsem0, gsem1, gsem2, gsem3)
        wsem = (wsem0, wsem1, wsem2, wsem3)

        pltpu.sync_copy(ids_hbm.at[pl.ds(base, B_PER_W)], idx_v)

        def gather(c):
            b = c % NBUF
            return pltpu.async_copy(
                table_hbm.at[idx_v.at[pl.ds(c * CH, CH)]], rows[b], gsem[b]
            )

        def writeback(c):
            b = c % NBUF
            return pltpu.async_copy(
                rows[b], out_hbm.at[pl.ds(base + c * CH, CH)], wsem[b]
            )

        def normalize(c):
            b = c % NBUF
            buf = rows[b]

            def row_body(r, carry):
                xs = [buf[r, pl.ds(L * j, L)] for j in range(NVREG)]
                acc = xs[0] * xs[0]
                for j in range(1, NVREG):
                    acc = acc + xs[j] * xs[j]
                t = jnp.full((L,), jnp.sum(acc) * (1.0 / HIDDEN) + EPS,
                             jnp.float32)
                yi = jnp.full((L,), 0x5F3759DF, jnp.int32) - \
                    lax.shift_right_logical(plsc.bitcast(t, jnp.int32), 1)
                y = plsc.bitcast(yi, jnp.float32)
                for _ in range(2):
                    y = y * (1.5 - 0.5 * t * y * y)
                for j in range(NVREG):
                    buf[r, pl.ds(L * j, L)] = xs[j] * y * wv[j]
                return carry

            lax.fori_loop(0, CH, row_body, 0, unroll=8)

        pending_g = [None] * NBUF
        pending_w = [None] * NBUF
        for p in range(PREFETCH):
            pending_g[p % NBUF] = gather(p)
        for c in range(NCHUNK):
            b = c % NBUF
            if c + PREFETCH < NCHUNK:
                tb = (c + PREFETCH) % NBUF
                if pending_w[tb] is not None:
                    pending_w[tb].wait()
                    pending_w[tb] = None
                pending_g[tb] = gather(c + PREFETCH)
            pending_g[b].wait()
            normalize(c)
            pending_w[b] = writeback(c)
        for d in pending_w:
            if d is not None:
                d.wait()

    return kern


_kern = _make_kernel()


def kernel(input_ids, table, ln_weight):
    ids = input_ids.reshape(-1).astype(jnp.int32)
    out = _kern(ids, table, ln_weight)
    return out.reshape(input_ids.shape + (HIDDEN,))


# speedup vs baseline: 1.0329x; 1.0329x over previous
import functools

import jax
import jax.numpy as jnp
from jax import lax
from jax.experimental import pallas as pl
from jax.experimental.pallas import tpu as pltpu
from jax.experimental.pallas import tpu_sc as plsc

HIDDEN = 128
EPS = 1e-05

NC = 2
NS = 16
L = 16
NW = NC * NS

B_TOTAL = 4 * 8192
B_PER_W = B_TOTAL // NW
CH = 256
GSPLIT = 128
NCHUNK = B_PER_W // CH
NVREG = HIDDEN // L
NBUF = 3
PREFETCH = 1


def _make_kernel():
    mesh = plsc.VectorSubcoreMesh(core_axis_name="c", subcore_axis_name="s")

    @functools.partial(
        pl.kernel,
        mesh=mesh,
        out_type=jax.ShapeDtypeStruct((B_TOTAL, HIDDEN), jnp.float32),
        scratch_types=[
            pltpu.VMEM((B_PER_W,), jnp.int32),
            pltpu.VMEM((CH, HIDDEN), jnp.float32),
            pltpu.VMEM((CH, HIDDEN), jnp.float32),
            pltpu.VMEM((CH, HIDDEN), jnp.float32),
            pltpu.VMEM((CH,), jnp.float32),
            pltpu.VMEM((HIDDEN,), jnp.float32),
            pltpu.SemaphoreType.DMA,
            pltpu.SemaphoreType.DMA,
            pltpu.SemaphoreType.DMA,
            pltpu.SemaphoreType.DMA,
            pltpu.SemaphoreType.DMA,
            pltpu.SemaphoreType.DMA,
        ],
        compiler_params=pltpu.CompilerParams(needs_layout_passes=False),
    )
    def kern(ids_hbm, table_hbm, w_hbm, out_hbm, idx_v, rows0, rows1, rows2,
             scale_v, w_v, gsem0, gsem1, gsem2, wsem0, wsem1, wsem2):
        wid = lax.axis_index("s") * NC + lax.axis_index("c")
        base = wid * B_PER_W
        pltpu.sync_copy(w_hbm, w_v)
        wv = [w_v[pl.ds(L * j, L)] for j in range(NVREG)]
        lane = lax.iota(jnp.int32, L)
        rows = (rows0, rows1, rows2)
        gsem = (gsem0, gsem1, gsem2)
        wsem = (wsem0, wsem1, wsem2)

        pltpu.sync_copy(ids_hbm.at[pl.ds(base, B_PER_W)], idx_v)

        def gather(c):
            b = c % NBUF
            return [
                pltpu.async_copy(
                    table_hbm.at[idx_v.at[pl.ds(c * CH + p * GSPLIT, GSPLIT)]],
                    rows[b].at[pl.ds(p * GSPLIT, GSPLIT)],
                    gsem[b],
                )
                for p in range(CH // GSPLIT)
            ]

        def writeback(c):
            b = c % NBUF
            return pltpu.async_copy(
                rows[b], out_hbm.at[pl.ds(base + c * CH, CH)], wsem[b]
            )

        def normalize(c):
            b = c % NBUF
            buf = rows[b]

            def group_sums(g, carry):
                r0 = g * L
                ms = jnp.zeros((L,), jnp.float32)
                for i in range(L):
                    r = r0 + i
                    xs = [buf[r, pl.ds(L * j, L)] for j in range(NVREG)]
                    sq = [x * x for x in xs]
                    acc = ((sq[0] + sq[1]) + (sq[2] + sq[3])) + (
                        (sq[4] + sq[5]) + (sq[6] + sq[7])
                    )
                    ms = jnp.where(lane == i, jnp.sum(acc), ms)
                t = ms * (1.0 / HIDDEN) + EPS
                yi = jnp.full((L,), 0x5F3759DF, jnp.int32) - \
                    lax.shift_right_logical(plsc.bitcast(t, jnp.int32), 1)
                y = plsc.bitcast(yi, jnp.float32)
                for _ in range(2):
                    y = y * (1.5 - 0.5 * t * y * y)
                scale_v[pl.ds(r0, L)] = y
                return carry

            def group_scale(g, carry):
                r0 = g * L
                y = scale_v[pl.ds(r0, L)]
                for i in range(L):
                    r = r0 + i
                    s = jnp.full((L,), y[i], jnp.float32)
                    for j in range(NVREG):
                        buf[r, pl.ds(L * j, L)] = (
                            buf[r, pl.ds(L * j, L)] * s * wv[j]
                        )
                return carry

            lax.fori_loop(0, CH // L, group_sums, 0)
            lax.fori_loop(0, CH // L, group_scale, 0)

        pending_g = [None] * NBUF
        pending_w = [None] * NBUF
        for p in range(PREFETCH):
            pending_g[p % NBUF] = gather(p)
        for c in range(NCHUNK):
            b = c % NBUF
            if c + PREFETCH < NCHUNK:
                tb = (c + PREFETCH) % NBUF
                if pending_w[tb] is not None:
                    pending_w[tb].wait()
                    pending_w[tb] = None
                pending_g[tb] = gather(c + PREFETCH)
            for d in pending_g[b]:
                d.wait()
            normalize(c)
            pending_w[b] = writeback(c)
        for d in pending_w:
            if d is not None:
                d.wait()

    return kern


_kern = _make_kernel()


def kernel(input_ids, table, ln_weight):
    ids = input_ids.reshape(-1).astype(jnp.int32)
    out = _kern(ids, table, ln_weight)
    return out.reshape(input_ids.shape + (HIDDEN,))


# speedup vs baseline: 1.0924x; 1.0576x over previous
import functools

import jax
import jax.numpy as jnp
from jax import lax
from jax.experimental import pallas as pl
from jax.experimental.pallas import tpu as pltpu
from jax.experimental.pallas import tpu_sc as plsc

HIDDEN = 128
EPS = 1e-05

NC = 2
NS = 16
L = 16
NW = NC * NS

BATCH = 4
SEQ = 8192
B_TOTAL = BATCH * SEQ
B_PER_W = B_TOTAL // NW
W_PER_BATCH = SEQ // B_PER_W
CH = 256
GSPLIT = 128
NCHUNK = B_PER_W // CH
NVREG = HIDDEN // L
NBUF = 3
PREFETCH = 1
QR = 4


def _make_kernel():
    mesh = plsc.VectorSubcoreMesh(core_axis_name="c", subcore_axis_name="s")

    @functools.partial(
        pl.kernel,
        mesh=mesh,
        out_type=jax.ShapeDtypeStruct((BATCH, SEQ, HIDDEN), jnp.float32),
        scratch_types=[
            pltpu.VMEM((B_PER_W,), jnp.int32),
            pltpu.VMEM((CH, HIDDEN), jnp.float32),
            pltpu.VMEM((CH, HIDDEN), jnp.float32),
            pltpu.VMEM((CH, HIDDEN), jnp.float32),
            pltpu.VMEM((HIDDEN,), jnp.float32),
            pltpu.SemaphoreType.DMA,
            pltpu.SemaphoreType.DMA,
            pltpu.SemaphoreType.DMA,
            pltpu.SemaphoreType.DMA,
            pltpu.SemaphoreType.DMA,
            pltpu.SemaphoreType.DMA,
        ],
        compiler_params=pltpu.CompilerParams(needs_layout_passes=False),
    )
    def kern(ids_hbm, table_hbm, w_hbm, out_hbm, idx_v, rows0, rows1, rows2,
             w_v, gsem0, gsem1, gsem2, wsem0, wsem1, wsem2):
        wid = lax.axis_index("s") * NC + lax.axis_index("c")
        batch = wid // W_PER_BATCH
        seq0 = (wid % W_PER_BATCH) * B_PER_W
        pltpu.sync_copy(w_hbm, w_v)
        wv = [w_v[pl.ds(L * j, L)] for j in range(NVREG)]
        lane = lax.iota(jnp.int32, L)
        rows = (rows0, rows1, rows2)
        gsem = (gsem0, gsem1, gsem2)
        wsem = (wsem0, wsem1, wsem2)

        pltpu.sync_copy(
            ids_hbm.at[batch, pl.ds(seq0, B_PER_W)], idx_v
        )

        def gather(c):
            b = c % NBUF
            return [
                pltpu.async_copy(
                    table_hbm.at[idx_v.at[pl.ds(c * CH + p * GSPLIT, GSPLIT)]],
                    rows[b].at[pl.ds(p * GSPLIT, GSPLIT)],
                    gsem[b],
                )
                for p in range(CH // GSPLIT)
            ]

        def writeback(c):
            b = c % NBUF
            return pltpu.async_copy(
                rows[b], out_hbm.at[batch, pl.ds(seq0 + c * CH, CH)], wsem[b]
            )

        def normalize(c):
            b = c % NBUF
            buf = rows[b]

            def quad_body(q, carry):
                r0 = q * QR
                xs = [
                    [buf[r0 + i, pl.ds(L * j, L)] for j in range(NVREG)]
                    for i in range(QR)
                ]
                ms = jnp.zeros((L,), jnp.float32)
                for i in range(QR):
                    sq = [x * x for x in xs[i]]
                    acc = ((sq[0] + sq[1]) + (sq[2] + sq[3])) + (
                        (sq[4] + sq[5]) + (sq[6] + sq[7])
                    )
                    ms = jnp.where(lane == i, jnp.sum(acc), ms)
                t = ms * (1.0 / HIDDEN) + EPS
                yi = jnp.full((L,), 0x5F3759DF, jnp.int32) - \
                    lax.shift_right_logical(plsc.bitcast(t, jnp.int32), 1)
                y = plsc.bitcast(yi, jnp.float32)
                for _ in range(2):
                    y = y * (1.5 - 0.5 * t * y * y)
                for i in range(QR):
                    s = jnp.full((L,), y[i], jnp.float32)
                    for j in range(NVREG):
                        buf[r0 + i, pl.ds(L * j, L)] = xs[i][j] * s * wv[j]
                return carry

            lax.fori_loop(0, CH // QR, quad_body, 0, unroll=2)

        pending_g = [None] * NBUF
        pending_w = [None] * NBUF
        for p in range(PREFETCH):
            pending_g[p % NBUF] = gather(p)
        for c in range(NCHUNK):
            b = c % NBUF
            if c + PREFETCH < NCHUNK:
                tb = (c + PREFETCH) % NBUF
                if pending_w[tb] is not None:
                    pending_w[tb].wait()
                    pending_w[tb] = None
                pending_g[tb] = gather(c + PREFETCH)
            for d in pending_g[b]:
                d.wait()
            normalize(c)
            pending_w[b] = writeback(c)
        for d in pending_w:
            if d is not None:
                d.wait()

    return kern


_kern = _make_kernel()


def kernel(input_ids, table, ln_weight):
    ids = input_ids.astype(jnp.int32)
    return _kern(ids, table, ln_weight)


# speedup vs baseline: 1.2036x; 1.1018x over previous
import functools

import jax
import jax.numpy as jnp
from jax import lax
from jax.experimental import pallas as pl
from jax.experimental.pallas import tpu as pltpu
from jax.experimental.pallas import tpu_sc as plsc

HIDDEN = 128
EPS = 1e-05

NC = 2
NS = 16
L = 16
NW = NC * NS

BATCH = 4
SEQ = 8192
B_TOTAL = BATCH * SEQ
B_PER_W = B_TOTAL // NW
W_PER_BATCH = SEQ // B_PER_W
CH = 128
NCHUNK = B_PER_W // CH
NVREG = HIDDEN // L
NBUF = 4
PREFETCH = 2
QR = 4


def _make_kernel():
    mesh = plsc.VectorSubcoreMesh(core_axis_name="c", subcore_axis_name="s")

    @functools.partial(
        pl.kernel,
        mesh=mesh,
        out_type=jax.ShapeDtypeStruct((BATCH, SEQ, HIDDEN), jnp.float32),
        scratch_types=[
            pltpu.VMEM((B_PER_W,), jnp.int32),
            pltpu.VMEM((NBUF * CH, HIDDEN), jnp.float32),
            pltpu.VMEM((HIDDEN,), jnp.float32),
            pltpu.SemaphoreType.DMA((NBUF,)),
            pltpu.SemaphoreType.DMA((NBUF,)),
        ],
        compiler_params=pltpu.CompilerParams(needs_layout_passes=False),
    )
    def kern(ids_hbm, table_hbm, w_hbm, out_hbm, idx_v, rows_v, w_v, gsem,
             wsem):
        wid = lax.axis_index("s") * NC + lax.axis_index("c")
        batch = wid // W_PER_BATCH
        seq0 = (wid % W_PER_BATCH) * B_PER_W
        pltpu.sync_copy(w_hbm, w_v)
        wv = [w_v[pl.ds(L * j, L)] for j in range(NVREG)]
        lane = lax.iota(jnp.int32, L)

        pltpu.sync_copy(ids_hbm.at[batch, pl.ds(seq0, B_PER_W)], idx_v)

        def gather_desc(c):
            b = lax.rem(c, NBUF)
            return pltpu.make_async_copy(
                table_hbm.at[idx_v.at[pl.ds(pl.multiple_of(c * CH, CH), CH)]],
                rows_v.at[pl.ds(pl.multiple_of(b * CH, CH), CH)],
                gsem.at[b],
            )

        def writeback_desc(c):
            b = lax.rem(c, NBUF)
            return pltpu.make_async_copy(
                rows_v.at[pl.ds(pl.multiple_of(b * CH, CH), CH)],
                out_hbm.at[batch, pl.ds(seq0 + c * CH, CH)],
                wsem.at[b],
            )

        def normalize(base):
            def quad_body(q, carry):
                r0 = base + q * QR
                xs = [
                    [rows_v[r0 + i, pl.ds(L * j, L)] for j in range(NVREG)]
                    for i in range(QR)
                ]
                ms = jnp.zeros((L,), jnp.float32)
                for i in range(QR):
                    sq = [x * x for x in xs[i]]
                    acc = ((sq[0] + sq[1]) + (sq[2] + sq[3])) + (
                        (sq[4] + sq[5]) + (sq[6] + sq[7])
                    )
                    ms = jnp.where(lane == i, jnp.sum(acc), ms)
                t = ms * (1.0 / HIDDEN) + EPS
                yi = jnp.full((L,), 0x5F3759DF, jnp.int32) - \
                    lax.shift_right_logical(plsc.bitcast(t, jnp.int32), 1)
                y = plsc.bitcast(yi, jnp.float32)
                for _ in range(2):
                    y = y * (1.5 - 0.5 * t * y * y)
                for i in range(QR):
                    s = jnp.full((L,), y[i], jnp.float32)
                    for j in range(NVREG):
                        rows_v[r0 + i, pl.ds(L * j, L)] = xs[i][j] * s * wv[j]
                return carry

            lax.fori_loop(0, CH // QR, quad_body, 0, unroll=2)

        for p in range(PREFETCH):
            gather_desc(jnp.int32(p)).start()

        def chunk_body(c, carry):
            b = lax.rem(c, NBUF)

            @pl.when(c + PREFETCH < NCHUNK)
            def _():
                @pl.when(c >= NBUF - PREFETCH)
                def _():
                    writeback_desc(c - (NBUF - PREFETCH)).wait()

                gather_desc(c + PREFETCH).start()

            gather_desc(c).wait()
            normalize(pl.multiple_of(b * CH, CH))
            writeback_desc(c).start()
            return carry

        lax.fori_loop(0, NCHUNK, chunk_body, jnp.int32(0))

        for c in range(NCHUNK - NBUF, NCHUNK):
            writeback_desc(jnp.int32(c)).wait()

    return kern


_kern = _make_kernel()


def kernel(input_ids, table, ln_weight):
    ids = input_ids.astype(jnp.int32)
    return _kern(ids, table, ln_weight)
